# baseline (speedup 1.0000x reference)
import functools

import jax
import jax.numpy as jnp
from jax import lax
from jax.experimental import pallas as pl
from jax.experimental.pallas import tpu as pltpu
from jax.experimental.pallas import tpu_sc as plsc

_B = 512
_C = 8192
_F = 2048
_K = 20
_TEMP = 0.05
_TEMP_DIST = 0.05
_ALPHA = 0.9

_NCE_CW = 512
_NCE_NC = _C // _NCE_CW
_ROW_BLK = 128
_MAIN_CW = 512
_MAIN_NC = _C // _MAIN_CW


def _nce_body(x_ref, t_ref, feat_ref, out_ref, xs, ms, ss, ts):
    i = pl.program_id(0)

    @pl.when(i == 0)
    def _init():
        x = x_ref[...]
        inv = lax.rsqrt(jnp.sum(x * x, axis=1, keepdims=True))
        xs[...] = x * inv
        ms[...] = jnp.full((_B, 1), -1e30, jnp.float32)
        ss[...] = jnp.zeros((_B, 1), jnp.float32)
        ts[...] = jnp.zeros((_B, 1), jnp.float32)

    blk = lax.dot_general(
        xs[...], feat_ref[...], (((1,), (1,)), ((), ())),
        preferred_element_type=jnp.float32) * (1.0 / _TEMP)
    m_old = ms[...]
    m_new = jnp.maximum(m_old, jnp.max(blk, axis=1, keepdims=True))
    ss[...] = (ss[...] * jnp.exp(m_old - m_new)
               + jnp.sum(jnp.exp(blk - m_new), axis=1, keepdims=True))
    ms[...] = m_new
    col = i * _NCE_CW + lax.broadcasted_iota(jnp.int32, (_B, _NCE_CW), 1)
    ts[...] += jnp.sum(jnp.where(col == t_ref[...], blk, 0.0), axis=1,
                       keepdims=True)

    @pl.when(i == _NCE_NC - 1)
    def _fin():
        lse = ms[...] + jnp.log(ss[...])
        out_ref[...] = (jnp.sum(lse - ts[...]) * (1.0 / _B)).reshape(1, 1)


def _stats_body(l0_ref, l1_ref, t_ref, lse0_ref, lse1_ref, ce_ref):
    i = pl.program_id(0)
    l0 = l0_ref[...]
    m0 = jnp.max(l0, axis=1, keepdims=True)
    lse0 = m0 + jnp.log(jnp.sum(jnp.exp(l0 - m0), axis=1, keepdims=True))
    lse0_ref[...] = lse0
    l1 = l1_ref[...]
    m1 = jnp.max(l1, axis=1, keepdims=True)
    lse1_ref[...] = m1 + jnp.log(
        jnp.sum(jnp.exp(l1 - m1), axis=1, keepdims=True))
    col = lax.broadcasted_iota(jnp.int32, (_ROW_BLK, _C), 1)
    tgt = jnp.sum(jnp.where(col == t_ref[...], l0, 0.0), axis=1, keepdims=True)

    @pl.when(i == 0)
    def _init():
        ce_ref[...] = jnp.zeros((1, 1), jnp.float32)

    ce_ref[...] += jnp.sum(tgt - lse0)


_SC_ROWS = 16


def _am_sc_body(nbt_hbm, dt_hbm, a_hbm, m_hbm, nbbuf, dbuf, abuf, mbuf):
    wid = lax.axis_index("s") * 2 + lax.axis_index("c")
    base = wid * _SC_ROWS
    pltpu.sync_copy(nbt_hbm.at[wid], nbbuf)
    pltpu.sync_copy(dt_hbm.at[wid], dbuf)
    lanes = lax.iota(jnp.int32, 16)
    zero16 = jnp.zeros((16,), jnp.float32)
    for j in range(_SC_ROWS * _B // 16):
        abuf[pl.ds(j * 16, 16)] = zero16
        mbuf[pl.ds(j * 16, 16)] = zero16
    ssum = jnp.zeros((16,), jnp.float32)
    for k in range(_K):
        ssum = ssum + jnp.exp(dbuf[k] * (1.0 / _TEMP_DIST))
    half_inv = 0.5 / ssum
    for k in range(_K):
        flat = lanes * _B + nbbuf[k]
        wk = jnp.exp(dbuf[k] * (1.0 / _TEMP_DIST)) * half_inv
        plsc.addupdate_scatter(abuf, [flat], wk)
        plsc.addupdate_scatter(mbuf, [flat],
                               jnp.full((16,), 1.0 / _K, jnp.float32))
    pltpu.sync_copy(abuf, a_hbm.at[pl.ds(base * _B, _SC_ROWS * _B)])
    pltpu.sync_copy(mbuf, m_hbm.at[pl.ds(base * _B, _SC_ROWS * _B)])


_am_sc = functools.partial(
    pl.kernel,
    mesh=plsc.VectorSubcoreMesh(core_axis_name="c", subcore_axis_name="s"),
    compiler_params=pltpu.CompilerParams(needs_layout_passes=False),
    out_type=[
        jax.ShapeDtypeStruct((_B * _B,), jnp.float32),
        jax.ShapeDtypeStruct((_B * _B,), jnp.float32),
    ],
    scratch_types=[
        pltpu.VMEM((_K, _SC_ROWS), jnp.int32),
        pltpu.VMEM((_K, _SC_ROWS), jnp.float32),
        pltpu.VMEM((_SC_ROWS * _B,), jnp.float32),
        pltpu.VMEM((_SC_ROWS * _B,), jnp.float32),
    ],
)(_am_sc_body)


def _main_body(l0_ref, l1_ref, lse0_ref, lse1_ref, a_ref, m_ref,
               acc1_ref, acc2_ref, acc3_ref):
    i = pl.program_id(0)
    lsm0 = l0_ref[...] - lse0_ref[...]
    p0 = jnp.exp(lsm0)
    lsm1 = l1_ref[...] - lse1_ref[...]
    p1 = jnp.exp(lsm1)
    t = lax.dot_general(m_ref[...], p0, (((1,), (0,)), ((), ())),
                        preferred_element_type=jnp.float32)
    s = lax.dot_general(a_ref[...], p0 + p1, (((1,), (0,)), ((), ())),
                        preferred_element_type=jnp.float32)
    tlogt = jnp.where(t > 0.0, t * jnp.log(jnp.where(t > 0.0, t, 1.0)), 0.0)

    @pl.when(i == 0)
    def _init():
        acc1_ref[...] = jnp.zeros((1, 1), jnp.float32)
        acc2_ref[...] = jnp.zeros((1, 1), jnp.float32)
        acc3_ref[...] = jnp.zeros((1, 1), jnp.float32)

    acc1_ref[...] += jnp.sum(tlogt)
    acc2_ref[...] += jnp.sum(t * lsm1)
    acc3_ref[...] += jnp.sum(s * lsm0)


def kernel(inputs0, logits0, logits1, targets, indexes, neighbors,
           neighbor_dists, rampup, features):
    del indexes
    t2d = targets.reshape(_B, 1)

    loss_nce = pl.pallas_call(
        _nce_body,
        grid=(_NCE_NC,),
        in_specs=[
            pl.BlockSpec((_B, _F), lambda i: (0, 0)),
            pl.BlockSpec((_B, 1), lambda i: (0, 0)),
            pl.BlockSpec((_NCE_CW, _F), lambda i: (i, 0)),
        ],
        out_specs=pl.BlockSpec((1, 1), lambda i: (0, 0)),
        out_shape=jax.ShapeDtypeStruct((1, 1), jnp.float32),
        scratch_shapes=[
            pltpu.VMEM((_B, _F), jnp.float32),
            pltpu.VMEM((_B, 1), jnp.float32),
            pltpu.VMEM((_B, 1), jnp.float32),
            pltpu.VMEM((_B, 1), jnp.float32),
        ],
    )(inputs0, t2d, features)

    lse0, lse1, ce_tgt = pl.pallas_call(
        _stats_body,
        grid=(_B // _ROW_BLK,),
        in_specs=[
            pl.BlockSpec((_ROW_BLK, _C), lambda i: (i, 0)),
            pl.BlockSpec((_ROW_BLK, _C), lambda i: (i, 0)),
            pl.BlockSpec((_ROW_BLK, 1), lambda i: (i, 0)),
        ],
        out_specs=[
            pl.BlockSpec((_ROW_BLK, 1), lambda i: (i, 0)),
            pl.BlockSpec((_ROW_BLK, 1), lambda i: (i, 0)),
            pl.BlockSpec((1, 1), lambda i: (0, 0)),
        ],
        out_shape=[
            jax.ShapeDtypeStruct((_B, 1), jnp.float32),
            jax.ShapeDtypeStruct((_B, 1), jnp.float32),
            jax.ShapeDtypeStruct((1, 1), jnp.float32),
        ],
    )(logits0, logits1, t2d)

    nb_t = neighbors.T.reshape(_K, 32, _SC_ROWS).transpose(1, 0, 2)
    d_t = neighbor_dists.T.reshape(_K, 32, _SC_ROWS).transpose(1, 0, 2)
    a_flat, m_flat = _am_sc(nb_t, d_t)
    a_mat = a_flat.reshape(_B, _B)
    m_mat = m_flat.reshape(_B, _B)

    acc1, acc2, acc3 = pl.pallas_call(
        _main_body,
        grid=(_MAIN_NC,),
        in_specs=[
            pl.BlockSpec((_B, _MAIN_CW), lambda i: (0, i)),
            pl.BlockSpec((_B, _MAIN_CW), lambda i: (0, i)),
            pl.BlockSpec((_B, 1), lambda i: (0, 0)),
            pl.BlockSpec((_B, 1), lambda i: (0, 0)),
            pl.BlockSpec((_B, _B), lambda i: (0, 0)),
            pl.BlockSpec((_B, _B), lambda i: (0, 0)),
        ],
        out_specs=[
            pl.BlockSpec((1, 1), lambda i: (0, 0)),
            pl.BlockSpec((1, 1), lambda i: (0, 0)),
            pl.BlockSpec((1, 1), lambda i: (0, 0)),
        ],
        out_shape=[
            jax.ShapeDtypeStruct((1, 1), jnp.float32),
            jax.ShapeDtypeStruct((1, 1), jnp.float32),
            jax.ShapeDtypeStruct((1, 1), jnp.float32),
        ],
    )(logits0, logits1, lse0, lse1, a_mat, m_mat)

    loss_ce = -(_ALPHA * ce_tgt[0, 0] + (1.0 - _ALPHA) * acc3[0, 0]) / _B
    loss_kl = (acc1[0, 0] - acc2[0, 0]) / _B
    return (loss_nce[0, 0], loss_ce, rampup * loss_kl)


# speedup vs baseline: 1.0446x; 1.0446x over previous
import functools

import jax
import jax.numpy as jnp
from jax import lax
from jax.experimental import pallas as pl
from jax.experimental.pallas import tpu as pltpu
from jax.experimental.pallas import tpu_sc as plsc

_B = 512
_C = 8192
_F = 2048
_K = 20
_TEMP = 0.05
_TEMP_DIST = 0.05
_ALPHA = 0.9

_CW = 512
_NC = _C // _CW
_NEG = -1e30

_SC_ROWS = 16


def _am_sc_body(nbt_hbm, dt_hbm, a_hbm, m_hbm, nbbuf, dbuf, abuf, mbuf):
    wid = lax.axis_index("s") * 2 + lax.axis_index("c")
    base = wid * _SC_ROWS
    pltpu.sync_copy(nbt_hbm.at[wid], nbbuf)
    pltpu.sync_copy(dt_hbm.at[wid], dbuf)
    lanes = lax.iota(jnp.int32, 16)
    zero16 = jnp.zeros((16,), jnp.float32)
    for j in range(_SC_ROWS * _B // 16):
        abuf[pl.ds(j * 16, 16)] = zero16
        mbuf[pl.ds(j * 16, 16)] = zero16
    ssum = jnp.zeros((16,), jnp.float32)
    for k in range(_K):
        ssum = ssum + jnp.exp(dbuf[k] * (1.0 / _TEMP_DIST))
    half_inv = 0.5 / ssum
    for k in range(_K):
        flat = lanes * _B + nbbuf[k]
        wk = jnp.exp(dbuf[k] * (1.0 / _TEMP_DIST)) * half_inv
        plsc.addupdate_scatter(abuf, [flat], wk)
        plsc.addupdate_scatter(mbuf, [flat],
                               jnp.full((16,), 1.0 / _K, jnp.float32))
    pltpu.sync_copy(abuf, a_hbm.at[pl.ds(base * _B, _SC_ROWS * _B)])
    pltpu.sync_copy(mbuf, m_hbm.at[pl.ds(base * _B, _SC_ROWS * _B)])


@functools.cache
def _am_sc_kernel():
    return functools.partial(
        pl.kernel,
        mesh=plsc.VectorSubcoreMesh(core_axis_name="c", subcore_axis_name="s"),
        compiler_params=pltpu.CompilerParams(needs_layout_passes=False),
        out_type=[
            jax.ShapeDtypeStruct((_B * _B,), jnp.float32),
            jax.ShapeDtypeStruct((_B * _B,), jnp.float32),
        ],
        scratch_types=[
            pltpu.VMEM((_K, _SC_ROWS), jnp.int32),
            pltpu.VMEM((_K, _SC_ROWS), jnp.float32),
            pltpu.VMEM((_SC_ROWS * _B,), jnp.float32),
            pltpu.VMEM((_SC_ROWS * _B,), jnp.float32),
        ],
    )(_am_sc_body)


def _build_am(nb_t, d_t):
    return _am_sc_kernel()(nb_t, d_t)


def _tc_body(x_ref, t_ref, feat_ref, l0_ref, l1_ref, a_ref, m_ref,
             nce_ref, ce_ref, kl_ref,
             xs, mo, so, to, m0, s0, m1, s1, tgt0, lse0, lse1, cet, acc3):
    j = pl.program_id(0)
    i = pl.program_id(1)

    @pl.when((j == 0) & (i == 0))
    def _init():
        x = x_ref[...]
        inv = lax.rsqrt(jnp.sum(x * x, axis=1, keepdims=True))
        xs[...] = (x * inv).astype(jnp.bfloat16)
        mo[...] = jnp.full((_B, 1), _NEG, jnp.float32)
        m0[...] = jnp.full((_B, 1), _NEG, jnp.float32)
        m1[...] = jnp.full((_B, 1), _NEG, jnp.float32)
        so[...] = jnp.zeros((_B, 1), jnp.float32)
        s0[...] = jnp.zeros((_B, 1), jnp.float32)
        s1[...] = jnp.zeros((_B, 1), jnp.float32)
        to[...] = jnp.zeros((_B, 1), jnp.float32)
        tgt0[...] = jnp.zeros((_B, 1), jnp.float32)
        acc3[...] = jnp.zeros((1, 1), jnp.float32)
        kl_ref[...] = jnp.zeros((1, 1), jnp.float32)

    col = i * _CW + lax.broadcasted_iota(jnp.int32, (_B, _CW), 1)
    hit = col == t_ref[...]

    @pl.when(j == 0)
    def _phase0():
        blk = lax.dot_general(
            xs[...], feat_ref[...].astype(jnp.bfloat16),
            (((1,), (1,)), ((), ())),
            preferred_element_type=jnp.float32) * (1.0 / _TEMP)
        m_old = mo[...]
        m_new = jnp.maximum(m_old, jnp.max(blk, axis=1, keepdims=True))
        so[...] = (so[...] * jnp.exp(m_old - m_new)
                   + jnp.sum(jnp.exp(blk - m_new), axis=1, keepdims=True))
        mo[...] = m_new
        to[...] += jnp.sum(jnp.where(hit, blk, 0.0), axis=1, keepdims=True)

        l0c = l0_ref[...]
        a_old = m0[...]
        a_new = jnp.maximum(a_old, jnp.max(l0c, axis=1, keepdims=True))
        s0[...] = (s0[...] * jnp.exp(a_old - a_new)
                   + jnp.sum(jnp.exp(l0c - a_new), axis=1, keepdims=True))
        m0[...] = a_new
        tgt0[...] += jnp.sum(jnp.where(hit, l0c, 0.0), axis=1, keepdims=True)

        l1c = l1_ref[...]
        b_old = m1[...]
        b_new = jnp.maximum(b_old, jnp.max(l1c, axis=1, keepdims=True))
        s1[...] = (s1[...] * jnp.exp(b_old - b_new)
                   + jnp.sum(jnp.exp(l1c - b_new), axis=1, keepdims=True))
        m1[...] = b_new

    @pl.when((j == 1) & (i == 0))
    def _finalize_stats():
        lse0[...] = m0[...] + jnp.log(s0[...])
        lse1[...] = m1[...] + jnp.log(s1[...])
        nce_ref[...] = (jnp.sum(mo[...] + jnp.log(so[...]) - to[...])
                        * (1.0 / _B)).reshape(1, 1)
        cet[...] = jnp.sum(tgt0[...] - lse0[...]).reshape(1, 1)

    @pl.when(j == 1)
    def _phase1():
        lsm0 = l0_ref[...] - lse0[...]
        p0 = jnp.exp(lsm0)
        lsm1 = l1_ref[...] - lse1[...]
        p1 = jnp.exp(lsm1)
        t = lax.dot_general(m_ref[...], p0, (((1,), (0,)), ((), ())),
                            preferred_element_type=jnp.float32)
        s = lax.dot_general(a_ref[...].astype(jnp.bfloat16),
                            (p0 + p1).astype(jnp.bfloat16),
                            (((1,), (0,)), ((), ())),
                            preferred_element_type=jnp.float32)
        kl_c = jnp.where(t > 0.0,
                         t * (jnp.log(jnp.where(t > 0.0, t, 1.0)) - lsm1),
                         0.0)
        kl_ref[...] += jnp.sum(kl_c)
        acc3[...] += jnp.sum(s * lsm0)

    @pl.when((j == 1) & (i == _NC - 1))
    def _finalize():
        ce_ref[...] = -(_ALPHA * cet[...]
                        + (1.0 - _ALPHA) * acc3[...]) * (1.0 / _B)
        kl_ref[...] = kl_ref[...] * (1.0 / _B)


def kernel(inputs0, logits0, logits1, targets, indexes, neighbors,
           neighbor_dists, rampup, features):
    del indexes
    t2d = targets.reshape(_B, 1)

    nb_t = neighbors.T.reshape(_K, 32, _SC_ROWS).transpose(1, 0, 2)
    d_t = neighbor_dists.T.reshape(_K, 32, _SC_ROWS).transpose(1, 0, 2)
    a_flat, m_flat = _build_am(nb_t, d_t)
    a_mat = a_flat.reshape(_B, _B)
    m_mat = m_flat.reshape(_B, _B)

    nce, ce, kl = pl.pallas_call(
        _tc_body,
        grid=(2, _NC),
        in_specs=[
            pl.BlockSpec((_B, _F), lambda j, i: (0, 0)),
            pl.BlockSpec((_B, 1), lambda j, i: (0, 0)),
            pl.BlockSpec((_CW, _F),
                         lambda j, i: (jnp.where(j == 0, i, _NC - 1), 0)),
            pl.BlockSpec((_B, _CW), lambda j, i: (0, i)),
            pl.BlockSpec((_B, _CW), lambda j, i: (0, i)),
            pl.BlockSpec((_B, _B), lambda j, i: (0, 0)),
            pl.BlockSpec((_B, _B), lambda j, i: (0, 0)),
        ],
        out_specs=[
            pl.BlockSpec((1, 1), lambda j, i: (0, 0)),
            pl.BlockSpec((1, 1), lambda j, i: (0, 0)),
            pl.BlockSpec((1, 1), lambda j, i: (0, 0)),
        ],
        out_shape=[
            jax.ShapeDtypeStruct((1, 1), jnp.float32),
            jax.ShapeDtypeStruct((1, 1), jnp.float32),
            jax.ShapeDtypeStruct((1, 1), jnp.float32),
        ],
        scratch_shapes=[
            pltpu.VMEM((_B, _F), jnp.bfloat16),
            pltpu.VMEM((_B, 1), jnp.float32),
            pltpu.VMEM((_B, 1), jnp.float32),
            pltpu.VMEM((_B, 1), jnp.float32),
            pltpu.VMEM((_B, 1), jnp.float32),
            pltpu.VMEM((_B, 1), jnp.float32),
            pltpu.VMEM((_B, 1), jnp.float32),
            pltpu.VMEM((_B, 1), jnp.float32),
            pltpu.VMEM((_B, 1), jnp.float32),
            pltpu.VMEM((_B, 1), jnp.float32),
            pltpu.VMEM((_B, 1), jnp.float32),
            pltpu.VMEM((1, 1), jnp.float32),
            pltpu.VMEM((1, 1), jnp.float32),
        ],
    )(inputs0, t2d, features, logits0, logits1, a_mat, m_mat)

    return (nce[0, 0], ce[0, 0], rampup * kl[0, 0])


# speedup vs baseline: 1.0929x; 1.0463x over previous
import functools

import jax
import jax.numpy as jnp
from jax import lax
from jax.experimental import pallas as pl
from jax.experimental.pallas import tpu as pltpu
from jax.experimental.pallas import tpu_sc as plsc

_B = 512
_C = 8192
_F = 2048
_K = 20
_TEMP = 0.05
_TEMP_DIST = 0.05
_ALPHA = 0.9

_CW = 512
_NC = _C // _CW
_NEG = -1e30

_SC_ROWS = 16


def _am_sc_body(nbt_hbm, dt_hbm, a_hbm, m_hbm, nbbuf, dbuf, abuf, mbuf):
    wid = lax.axis_index("s") * 2 + lax.axis_index("c")
    base = wid * _SC_ROWS
    pltpu.sync_copy(nbt_hbm.at[wid], nbbuf)
    pltpu.sync_copy(dt_hbm.at[wid], dbuf)
    lanes = lax.iota(jnp.int32, 16)
    zero16 = jnp.zeros((16,), jnp.float32)
    for j in range(_SC_ROWS * _B // 16):
        abuf[pl.ds(j * 16, 16)] = zero16
        mbuf[pl.ds(j * 16, 16)] = zero16
    ssum = jnp.zeros((16,), jnp.float32)
    for k in range(_K):
        ssum = ssum + jnp.exp(dbuf[k] * (1.0 / _TEMP_DIST))
    half_inv = 0.5 / ssum
    for k in range(_K):
        flat = lanes * _B + nbbuf[k]
        wk = jnp.exp(dbuf[k] * (1.0 / _TEMP_DIST)) * half_inv
        plsc.addupdate_scatter(abuf, [flat], wk)
        plsc.addupdate_scatter(mbuf, [flat],
                               jnp.full((16,), 1.0 / _K, jnp.float32))
    pltpu.sync_copy(abuf, a_hbm.at[pl.ds(base * _B, _SC_ROWS * _B)])
    pltpu.sync_copy(mbuf, m_hbm.at[pl.ds(base * _B, _SC_ROWS * _B)])


@functools.cache
def _am_sc_kernel():
    return functools.partial(
        pl.kernel,
        mesh=plsc.VectorSubcoreMesh(core_axis_name="c", subcore_axis_name="s"),
        compiler_params=pltpu.CompilerParams(needs_layout_passes=False),
        out_type=[
            jax.ShapeDtypeStruct((_B * _B,), jnp.float32),
            jax.ShapeDtypeStruct((_B * _B,), jnp.float32),
        ],
        scratch_types=[
            pltpu.VMEM((_K, _SC_ROWS), jnp.int32),
            pltpu.VMEM((_K, _SC_ROWS), jnp.float32),
            pltpu.VMEM((_SC_ROWS * _B,), jnp.float32),
            pltpu.VMEM((_SC_ROWS * _B,), jnp.float32),
        ],
    )(_am_sc_body)


def _build_am(nb_t, d_t):
    return _am_sc_kernel()(nb_t, d_t)


def _tc0_body(x_ref, t_ref, feat_ref, l0_ref, l1_ref,
              nce_ref, lse0_ref, lse1_ref, cet_ref,
              xs, mo, so, to, m0, s0, m1, s1, tgt0):
    i = pl.program_id(0)

    @pl.when(i == 0)
    def _init():
        x = x_ref[...]
        inv = lax.rsqrt(jnp.sum(x * x, axis=1, keepdims=True))
        xs[...] = (x * inv).astype(jnp.bfloat16)
        mo[...] = jnp.full((_B, 1), _NEG, jnp.float32)
        m0[...] = jnp.full((_B, 1), _NEG, jnp.float32)
        m1[...] = jnp.full((_B, 1), _NEG, jnp.float32)
        so[...] = jnp.zeros((_B, 1), jnp.float32)
        s0[...] = jnp.zeros((_B, 1), jnp.float32)
        s1[...] = jnp.zeros((_B, 1), jnp.float32)
        to[...] = jnp.zeros((_B, 1), jnp.float32)
        tgt0[...] = jnp.zeros((_B, 1), jnp.float32)

    col = i * _CW + lax.broadcasted_iota(jnp.int32, (_B, _CW), 1)
    hit = col == t_ref[...]

    if True:
        blk = lax.dot_general(
            xs[...], feat_ref[...].astype(jnp.bfloat16),
            (((1,), (1,)), ((), ())),
            preferred_element_type=jnp.float32) * (1.0 / _TEMP)
        m_old = mo[...]
        m_new = jnp.maximum(m_old, jnp.max(blk, axis=1, keepdims=True))
        so[...] = (so[...] * jnp.exp(m_old - m_new)
                   + jnp.sum(jnp.exp(blk - m_new), axis=1, keepdims=True))
        mo[...] = m_new
        to[...] += jnp.sum(jnp.where(hit, blk, 0.0), axis=1, keepdims=True)

        l0c = l0_ref[...]
        a_old = m0[...]
        a_new = jnp.maximum(a_old, jnp.max(l0c, axis=1, keepdims=True))
        s0[...] = (s0[...] * jnp.exp(a_old - a_new)
                   + jnp.sum(jnp.exp(l0c - a_new), axis=1, keepdims=True))
        m0[...] = a_new
        tgt0[...] += jnp.sum(jnp.where(hit, l0c, 0.0), axis=1, keepdims=True)

        l1c = l1_ref[...]
        b_old = m1[...]
        b_new = jnp.maximum(b_old, jnp.max(l1c, axis=1, keepdims=True))
        s1[...] = (s1[...] * jnp.exp(b_old - b_new)
                   + jnp.sum(jnp.exp(l1c - b_new), axis=1, keepdims=True))
        m1[...] = b_new

    @pl.when(i == _NC - 1)
    def _finalize_stats():
        lse0 = m0[...] + jnp.log(s0[...])
        lse1 = m1[...] + jnp.log(s1[...])
        lse0_ref[...] = lse0
        lse1_ref[...] = lse1
        nce_ref[...] = (jnp.sum(mo[...] + jnp.log(so[...]) - to[...])
                        * (1.0 / _B)).reshape(1, 1)
        cet_ref[...] = jnp.sum(tgt0[...] - lse0).reshape(1, 1)


def _tc1_body(l0_ref, l1_ref, a_ref, m_ref, lse0_ref, lse1_ref, cet_ref,
              ce_ref, kl_ref, acc3):
    i = pl.program_id(0)

    @pl.when(i == 0)
    def _init():
        acc3[...] = jnp.zeros((1, 1), jnp.float32)
        kl_ref[...] = jnp.zeros((1, 1), jnp.float32)

    if True:
        lsm0 = l0_ref[...] - lse0_ref[...]
        p0 = jnp.exp(lsm0)
        lsm1 = l1_ref[...] - lse1_ref[...]
        p1 = jnp.exp(lsm1)
        t = lax.dot_general(m_ref[...], p0, (((1,), (0,)), ((), ())),
                            preferred_element_type=jnp.float32)
        s = lax.dot_general(a_ref[...].astype(jnp.bfloat16),
                            (p0 + p1).astype(jnp.bfloat16),
                            (((1,), (0,)), ((), ())),
                            preferred_element_type=jnp.float32)
        kl_c = jnp.where(t > 0.0,
                         t * (jnp.log(jnp.where(t > 0.0, t, 1.0)) - lsm1),
                         0.0)
        kl_ref[...] += jnp.sum(kl_c)
        acc3[...] += jnp.sum(s * lsm0)

    @pl.when(i == _NC - 1)
    def _finalize():
        ce_ref[...] = -(_ALPHA * cet_ref[...]
                        + (1.0 - _ALPHA) * acc3[...]) * (1.0 / _B)
        kl_ref[...] = kl_ref[...] * (1.0 / _B)


def kernel(inputs0, logits0, logits1, targets, indexes, neighbors,
           neighbor_dists, rampup, features):
    del indexes
    t2d = targets.reshape(_B, 1)

    nb_t = neighbors.T.reshape(_K, 32, _SC_ROWS).transpose(1, 0, 2)
    d_t = neighbor_dists.T.reshape(_K, 32, _SC_ROWS).transpose(1, 0, 2)
    a_flat, m_flat = _build_am(nb_t, d_t)
    a_mat = a_flat.reshape(_B, _B)
    m_mat = m_flat.reshape(_B, _B)

    nce, lse0, lse1, cet = pl.pallas_call(
        _tc0_body,
        grid=(_NC,),
        in_specs=[
            pl.BlockSpec((_B, _F), lambda i: (0, 0)),
            pl.BlockSpec((_B, 1), lambda i: (0, 0)),
            pl.BlockSpec((_CW, _F), lambda i: (i, 0)),
            pl.BlockSpec((_B, _CW), lambda i: (0, i)),
            pl.BlockSpec((_B, _CW), lambda i: (0, i)),
        ],
        out_specs=[
            pl.BlockSpec((1, 1), lambda i: (0, 0)),
            pl.BlockSpec((_B, 1), lambda i: (0, 0)),
            pl.BlockSpec((_B, 1), lambda i: (0, 0)),
            pl.BlockSpec((1, 1), lambda i: (0, 0)),
        ],
        out_shape=[
            jax.ShapeDtypeStruct((1, 1), jnp.float32),
            jax.ShapeDtypeStruct((_B, 1), jnp.float32),
            jax.ShapeDtypeStruct((_B, 1), jnp.float32),
            jax.ShapeDtypeStruct((1, 1), jnp.float32),
        ],
        scratch_shapes=[
            pltpu.VMEM((_B, _F), jnp.bfloat16),
            pltpu.VMEM((_B, 1), jnp.float32),
            pltpu.VMEM((_B, 1), jnp.float32),
            pltpu.VMEM((_B, 1), jnp.float32),
            pltpu.VMEM((_B, 1), jnp.float32),
            pltpu.VMEM((_B, 1), jnp.float32),
            pltpu.VMEM((_B, 1), jnp.float32),
            pltpu.VMEM((_B, 1), jnp.float32),
            pltpu.VMEM((_B, 1), jnp.float32),
        ],
    )(inputs0, t2d, features, logits0, logits1)

    ce, kl = pl.pallas_call(
        _tc1_body,
        grid=(_NC,),
        in_specs=[
            pl.BlockSpec((_B, _CW), lambda i: (0, i)),
            pl.BlockSpec((_B, _CW), lambda i: (0, i)),
            pl.BlockSpec((_B, _B), lambda i: (0, 0)),
            pl.BlockSpec((_B, _B), lambda i: (0, 0)),
            pl.BlockSpec((_B, 1), lambda i: (0, 0)),
            pl.BlockSpec((_B, 1), lambda i: (0, 0)),
            pl.BlockSpec((1, 1), lambda i: (0, 0)),
        ],
        out_specs=[
            pl.BlockSpec((1, 1), lambda i: (0, 0)),
            pl.BlockSpec((1, 1), lambda i: (0, 0)),
        ],
        out_shape=[
            jax.ShapeDtypeStruct((1, 1), jnp.float32),
            jax.ShapeDtypeStruct((1, 1), jnp.float32),
        ],
        scratch_shapes=[
            pltpu.VMEM((1, 1), jnp.float32),
        ],
    )(logits0, logits1, a_mat, m_mat, lse0, lse1, cet)

    return (nce[0, 0], ce[0, 0], rampup * kl[0, 0])
